# Initial kernel scaffold; baseline (speedup 1.0000x reference)
import jax
import jax.numpy as jnp
from jax.experimental import pallas as pl


def kernel(x, edge_index, edge_weight, W_fc, b_gcn, W_S, b_S, W_P, b_P):
    raise NotImplementedError("write your pallas kernel here")


# speedup vs baseline: 4.5041x; 4.5041x over previous
import functools

import jax
import jax.numpy as jnp
from jax import lax
from jax.experimental import pallas as pl
from jax.experimental.pallas import tpu as pltpu
from jax.experimental.pallas import tpu_sc as plsc

NC = 2
NS = 16
NW = NC * NS
CHUNK = 128


def _sc_gather_scatter(x, ei, w, n_pad):
    _, fd = x.shape
    ep = w.shape[0]
    n_chunks = ep // CHUNK
    q, r = divmod(n_chunks, NW)
    rows_per_tile = n_pad // NS
    zrows = 128
    mesh = plsc.VectorSubcoreMesh(core_axis_name="c", subcore_axis_name="s",
                                  num_cores=NC, num_subcores=NS)

    @functools.partial(
        pl.kernel,
        mesh=mesh,
        out_type=jax.ShapeDtypeStruct((NC, n_pad, fd), jnp.float32),
        scratch_types=[
            pltpu.VMEM((CHUNK,), jnp.int32),
            pltpu.VMEM((CHUNK,), jnp.int32),
            pltpu.VMEM((CHUNK,), jnp.float32),
            pltpu.VMEM((CHUNK, fd), jnp.float32),
            pltpu.VMEM((zrows, fd), jnp.float32),
            pltpu.VMEM((zrows,), jnp.int32),
            pltpu.VMEM_SHARED((n_pad, fd), jnp.float32),
            pltpu.SemaphoreType.DMA,
        ],
    )
    def sc(x_hbm, ei_hbm, w_hbm, out_hbm,
           sidx_v, didx_v, w_v, rows_v, zbuf_v, ridx_v, acc_sh, sem):
        cid = lax.axis_index("c")
        sid = lax.axis_index("s")
        wid = sid * NC + cid

        def zrow(i, carry):
            rr = i // (fd // 16)
            c0 = (i % (fd // 16)) * 16
            zbuf_v[rr, pl.ds(c0, 16)] = jnp.zeros((16,), jnp.float32)
            return carry

        lax.fori_loop(0, zrows * (fd // 16), zrow, 0)
        iota16 = lax.iota(jnp.int32, 16)
        for k in range(rows_per_tile // zrows):
            base_row = sid * rows_per_tile + k * zrows
            for g in range(zrows // 16):
                ridx_v[pl.ds(g * 16, 16)] = iota16 + (base_row + g * 16)
            pltpu.sync_copy(zbuf_v, acc_sh.at[ridx_v])
        plsc.subcore_barrier()

        extra = jnp.minimum(wid, r)
        start = wid * q + extra
        count = q + jnp.where(wid < r, 1, 0)

        def chunk_body(c, carry):
            base = (start + c) * CHUNK
            pltpu.sync_copy(ei_hbm.at[0, pl.ds(base, CHUNK)], sidx_v)
            pltpu.sync_copy(ei_hbm.at[1, pl.ds(base, CHUNK)], didx_v)
            pltpu.sync_copy(w_hbm.at[pl.ds(base, CHUNK)], w_v)
            pltpu.async_copy(x_hbm.at[sidx_v], rows_v, sem).wait()

            def scale(g, carry2):
                wv16 = w_v[pl.ds(g * 16, 16)]
                for k in range(16):
                    wk = jnp.full((16,), wv16[k], jnp.float32)
                    i = g * 16 + k
                    for j in range(fd // 16):
                        rows_v[i, pl.ds(j * 16, 16)] = (
                            rows_v[i, pl.ds(j * 16, 16)] * wk)
                return carry2

            lax.fori_loop(0, CHUNK // 16, scale, 0)
            pltpu.sync_copy(rows_v, acc_sh.at[didx_v], add=True)
            return carry

        lax.fori_loop(0, count, chunk_body, 0)
        plsc.subcore_barrier()

        pltpu.sync_copy(
            acc_sh.at[pl.ds(sid * rows_per_tile, rows_per_tile)],
            out_hbm.at[cid, pl.ds(sid * rows_per_tile, rows_per_tile)])

    return sc(x, ei, w)


def _epilogue(p0, p1, z, w_fc, b_gcn, w_s, b_s, w_p, b_p):
    n, fd = p0.shape
    hd = w_fc.shape[1]
    c = w_s.shape[1]
    p = w_p.shape[1]
    bm = 1000

    def body(p0_ref, p1_ref, z_ref, wf_ref, bg_ref, ws_ref, bs_ref, wp_ref,
             bp_ref, co_ref, pr_ref):
        agg = p0_ref[...] + p1_ref[...] + z_ref[...] * 0.0
        tmp = jnp.maximum(
            jnp.dot(agg, wf_ref[...], preferred_element_type=jnp.float32,
                    precision=lax.Precision.HIGHEST) + bg_ref[...], 0.0)
        co_ref[...] = jnp.dot(tmp, ws_ref[...],
                              preferred_element_type=jnp.float32,
                              precision=lax.Precision.HIGHEST) + bs_ref[...]
        pr_ref[...] = jnp.dot(tmp, wp_ref[...],
                              preferred_element_type=jnp.float32,
                              precision=lax.Precision.HIGHEST) + bp_ref[...]

    return pl.pallas_call(
        body,
        grid=(n // bm,),
        in_specs=[
            pl.BlockSpec((bm, fd), lambda i: (i, 0)),
            pl.BlockSpec((bm, fd), lambda i: (i, 0)),
            pl.BlockSpec((1, 1), lambda i: (0, 0)),
            pl.BlockSpec((fd, hd), lambda i: (0, 0)),
            pl.BlockSpec((1, hd), lambda i: (0, 0)),
            pl.BlockSpec((hd, c), lambda i: (0, 0)),
            pl.BlockSpec((1, c), lambda i: (0, 0)),
            pl.BlockSpec((hd, p), lambda i: (0, 0)),
            pl.BlockSpec((1, p), lambda i: (0, 0)),
        ],
        out_specs=[
            pl.BlockSpec((bm, c), lambda i: (i, 0)),
            pl.BlockSpec((bm, p), lambda i: (i, 0)),
        ],
        out_shape=[
            jax.ShapeDtypeStruct((n, c), jnp.float32),
            jax.ShapeDtypeStruct((n, p), jnp.float32),
        ],
    )(p0, p1, z.reshape(1, 1), w_fc, b_gcn, w_s, b_s, w_p, b_p)


def kernel(x, edge_index, edge_weight, W_fc, b_gcn, W_S, b_S, W_P, b_P):
    n, fd = x.shape
    hd = W_fc.shape[1]
    e = edge_weight.shape[0]
    ei, w = edge_index, edge_weight
    if e % CHUNK:
        pad = CHUNK - e % CHUNK
        ei = jnp.pad(ei, ((0, 0), (0, pad)))
        w = jnp.pad(w, (0, pad))
        ei, w = lax.optimization_barrier((ei, w))

    n_pad = ((n + NS * 8 - 1) // (NS * 8)) * (NS * 8)
    parts = _sc_gather_scatter(x, ei, w, n_pad)
    z = jnp.sum(parts)
    p0, p1, z = lax.optimization_barrier((parts[0, :n], parts[1, :n], z))
    common, private = _epilogue(p0, p1, z, W_fc,
                                b_gcn.reshape(1, hd), W_S, b_S.reshape(1, -1),
                                W_P, b_P.reshape(1, -1))
    common, private, _ = lax.optimization_barrier((common, private, z))
    return (common, private)
